# baseline (speedup 1.0000x reference)
import functools

import jax
import jax.numpy as jnp
from jax import lax
from jax.experimental import pallas as pl
from jax.experimental.pallas import tpu as pltpu
from jax.experimental.pallas import tpu_sc as plsc

_NC, _NS = 2, 16
_NW = _NC * _NS
_BLK = 128


def _dot_t(a, b):
    return jax.lax.dot_general(a, b, (((1,), (1,)), ((), ())),
                               preferred_element_type=jnp.float32)


def _router_kernel(logits_ref, sel_ref, a1_ref, a2_ref, wv1_ref, wv2_ref):
    logits = logits_ref[...]
    t, e = logits.shape
    m = jnp.max(logits, axis=-1, keepdims=True)
    ex = jnp.exp(logits - m)
    scores = ex / jnp.sum(ex, axis=-1, keepdims=True)
    lane = jax.lax.broadcasted_iota(jnp.int32, (t, e), 1)
    s1 = jnp.max(scores, axis=-1, keepdims=True)
    a1 = jnp.min(jnp.where(scores == s1, lane, e), axis=-1, keepdims=True)
    m1 = lane == a1
    scores2 = jnp.where(m1, -1.0, scores)
    s2 = jnp.max(scores2, axis=-1, keepdims=True)
    a2 = jnp.min(jnp.where(scores2 == s2, lane, e), axis=-1, keepdims=True)
    m2 = lane == a2
    denom = s1 + s2 + 1e-6
    sel_ref[...] = jnp.where(m1 | m2, 1.0, 0.0)
    a1_ref[...] = a1
    a2_ref[...] = a2
    wv1_ref[...] = s1 / denom
    wv2_ref[...] = s2 / denom


def _pos_kernel(sel_ref, a1_ref, a2_ref, pos1_ref, pos2_ref, cnt_ref, carry):
    i = pl.program_id(0)
    blk = sel_ref[...]
    bt2, e = blk.shape

    @pl.when(i == 0)
    def _():
        carry[...] = jnp.zeros_like(carry)

    r = jax.lax.broadcasted_iota(jnp.int32, (bt2, bt2), 0)
    c = jax.lax.broadcasted_iota(jnp.int32, (bt2, bt2), 1)
    tri = jnp.where(r > c, 1.0, 0.0)
    pos = jax.lax.dot_general(tri, blk, (((1,), (0,)), ((), ())),
                              preferred_element_type=jnp.float32)
    pos = pos + carry[...]
    carry[...] += jnp.sum(blk, axis=0, keepdims=True)
    lane = jax.lax.broadcasted_iota(jnp.int32, (bt2, e), 1)
    pos1_ref[...] = jnp.sum(
        jnp.where(lane == a1_ref[...], pos, 0.0), axis=1, keepdims=True)
    pos2_ref[...] = jnp.sum(
        jnp.where(lane == a2_ref[...], pos, 0.0), axis=1, keepdims=True)

    @pl.when(i == pl.num_programs(0) - 1)
    def _():
        cnt_ref[...] = carry[...]


def _shared_kernel(x_ref, sw1_ref, sw3_ref, sw2_ref, o_ref):
    x = x_ref[...].astype(jnp.bfloat16)
    g = _dot_t(x, sw1_ref[...].astype(jnp.bfloat16))
    u = _dot_t(x, sw3_ref[...].astype(jnp.bfloat16))
    h = (g * jax.nn.sigmoid(g) * u).astype(jnp.bfloat16)
    o_ref[...] = _dot_t(h, sw2_ref[...].astype(jnp.bfloat16))


def _ffn_kernel(be_ref, xs_ref, w1_ref, w3_ref, w2_ref, rw_ref, ys_ref):
    del be_ref
    x = xs_ref[...].astype(jnp.bfloat16)
    g = _dot_t(x, w1_ref[0].astype(jnp.bfloat16))
    u = _dot_t(x, w3_ref[0].astype(jnp.bfloat16))
    h = (g * jax.nn.sigmoid(g) * u).astype(jnp.bfloat16)
    y = _dot_t(h, w2_ref[0].astype(jnp.bfloat16))
    rw = rw_ref[0, 0, :]
    ys_ref[...] = y * rw[:, None]


def _add3_kernel(a_ref, b_ref, c_ref, o_ref):
    o_ref[...] = a_ref[...] + b_ref[...] + c_ref[...]


def _sc_gather_rows(x, row_token, maxr):
    _, w = x.shape
    rows_pw = maxr // _NW
    nch = 4
    ch = rows_pw // nch
    mesh = plsc.VectorSubcoreMesh(core_axis_name="c", subcore_axis_name="s")

    @functools.partial(
        pl.kernel, mesh=mesh,
        out_type=jax.ShapeDtypeStruct((maxr, w), jnp.float32),
        scratch_types=[pltpu.VMEM((rows_pw,), jnp.int32),
                       pltpu.VMEM((ch, w), jnp.float32),
                       pltpu.VMEM((ch, w), jnp.float32),
                       pltpu.SemaphoreType.DMA,
                       pltpu.SemaphoreType.DMA])
    def k(x_hbm, tok_hbm, out_hbm, idx_v, b0, b1, s0, s1):
        wid = lax.axis_index("s") * _NC + lax.axis_index("c")
        base = wid * rows_pw
        pltpu.sync_copy(tok_hbm.at[pl.ds(base, rows_pw)], idx_v)
        bufs, sems, g = (b0, b1), (s0, s1), [None, None]
        g[0] = pltpu.async_copy(
            x_hbm.at[idx_v.at[pl.ds(0, ch)]], b0, s0)
        for i in range(nch):
            if i + 1 < nch:
                g[(i + 1) % 2] = pltpu.async_copy(
                    x_hbm.at[idx_v.at[pl.ds((i + 1) * ch, ch)]],
                    bufs[(i + 1) % 2], sems[(i + 1) % 2])
            g[i % 2].wait()
            pltpu.sync_copy(bufs[i % 2], out_hbm.at[pl.ds(base + i * ch, ch)])

    return k(x, row_token)


def _sc_gather2(ys, r1, r2):
    _, w = ys.shape
    t = r1.shape[0]
    toks_pw = t // _NW
    mesh = plsc.VectorSubcoreMesh(core_axis_name="c", subcore_axis_name="s")

    ch = toks_pw // 2

    @functools.partial(
        pl.kernel, mesh=mesh,
        out_type=(jax.ShapeDtypeStruct((t, w), jnp.float32),
                  jax.ShapeDtypeStruct((t, w), jnp.float32)),
        scratch_types=[pltpu.VMEM((toks_pw,), jnp.int32),
                       pltpu.VMEM((toks_pw,), jnp.int32),
                       pltpu.VMEM((ch, w), jnp.float32),
                       pltpu.VMEM((ch, w), jnp.float32),
                       pltpu.SemaphoreType.DMA,
                       pltpu.SemaphoreType.DMA])
    def k(ys_hbm, r1_hbm, r2_hbm, ysa_hbm, ysb_hbm,
          i1_v, i2_v, ba, bb, sa, sb):
        wid = lax.axis_index("s") * _NC + lax.axis_index("c")
        base = wid * toks_pw
        pltpu.sync_copy(r1_hbm.at[pl.ds(base, toks_pw)], i1_v)
        pltpu.sync_copy(r2_hbm.at[pl.ds(base, toks_pw)], i2_v)
        for i in range(toks_pw // ch):
            ga = pltpu.async_copy(
                ys_hbm.at[i1_v.at[pl.ds(i * ch, ch)]], ba, sa)
            gb = pltpu.async_copy(
                ys_hbm.at[i2_v.at[pl.ds(i * ch, ch)]], bb, sb)
            ga.wait()
            pltpu.sync_copy(ba, ysa_hbm.at[pl.ds(base + i * ch, ch)])
            gb.wait()
            pltpu.sync_copy(bb, ysb_hbm.at[pl.ds(base + i * ch, ch)])

    return k(ys, r1, r2)


def kernel(hidden_states, gate_w, w1, w2, w3, sw1, sw2, sw3):
    b, s, d = hidden_states.shape
    t = b * s
    e = gate_w.shape[0]
    df = w1.shape[1]
    dsh = sw1.shape[0]
    x = hidden_states.reshape(t, d)

    maxb = (t * 2) // _BLK + e
    maxr = maxb * _BLK

    logits = x @ gate_w.T
    selmask, a1, a2, wv1, wv2 = pl.pallas_call(
        _router_kernel,
        out_shape=(jax.ShapeDtypeStruct((t, e), jnp.float32),
                   jax.ShapeDtypeStruct((t, 1), jnp.int32),
                   jax.ShapeDtypeStruct((t, 1), jnp.int32),
                   jax.ShapeDtypeStruct((t, 1), jnp.float32),
                   jax.ShapeDtypeStruct((t, 1), jnp.float32)),
    )(logits)

    bt2 = 256
    pos1, pos2, cnt = pl.pallas_call(
        _pos_kernel,
        grid=(t // bt2,),
        in_specs=[
            pl.BlockSpec((bt2, e), lambda i: (i, 0)),
            pl.BlockSpec((bt2, 1), lambda i: (i, 0)),
            pl.BlockSpec((bt2, 1), lambda i: (i, 0)),
        ],
        out_specs=(pl.BlockSpec((bt2, 1), lambda i: (i, 0)),
                   pl.BlockSpec((bt2, 1), lambda i: (i, 0)),
                   pl.BlockSpec((1, e), lambda i: (0, 0))),
        out_shape=(jax.ShapeDtypeStruct((t, 1), jnp.float32),
                   jax.ShapeDtypeStruct((t, 1), jnp.float32),
                   jax.ShapeDtypeStruct((1, e), jnp.float32)),
        scratch_shapes=[pltpu.VMEM((1, e), jnp.float32)],
        compiler_params=pltpu.CompilerParams(
            dimension_semantics=("arbitrary",)),
    )(selmask, a1, a2)

    counts = cnt[0].astype(jnp.int32)
    pc = ((counts + _BLK - 1) // _BLK) * _BLK
    cum = jnp.cumsum(pc)
    poff = cum - pc
    bi = jnp.arange(maxb, dtype=jnp.int32) * _BLK
    block_expert = jnp.minimum(
        jnp.sum((bi[:, None] >= cum[None, :]).astype(jnp.int32), axis=1),
        e - 1).astype(jnp.int32)
    r1 = jnp.take(poff, a1[:, 0]) + pos1[:, 0].astype(jnp.int32)
    r2 = jnp.take(poff, a2[:, 0]) + pos2[:, 0].astype(jnp.int32)
    tok = jnp.arange(t, dtype=jnp.int32)
    pad_tok = (jnp.arange(maxr, dtype=jnp.int32) * 64) % t
    row_token = pad_tok.at[r1].set(tok).at[r2].set(tok)
    row_weight = jnp.zeros((maxr,), jnp.float32).at[r1].set(
        wv1[:, 0]).at[r2].set(wv2[:, 0])

    xs = _sc_gather_rows(x, row_token, maxr)

    bts = 512
    shared = pl.pallas_call(
        _shared_kernel,
        grid=(t // bts,),
        in_specs=[
            pl.BlockSpec((bts, d), lambda i: (i, 0)),
            pl.BlockSpec((dsh, d), lambda i: (0, 0)),
            pl.BlockSpec((dsh, d), lambda i: (0, 0)),
            pl.BlockSpec((d, dsh), lambda i: (0, 0)),
        ],
        out_specs=pl.BlockSpec((bts, d), lambda i: (i, 0)),
        out_shape=jax.ShapeDtypeStruct((t, d), jnp.float32),
    )(x, sw1, sw3, sw2)

    rw3d = row_weight.reshape(maxb, 1, _BLK)
    grid_spec = pltpu.PrefetchScalarGridSpec(
        num_scalar_prefetch=1,
        grid=(maxb,),
        in_specs=[
            pl.BlockSpec((_BLK, d), lambda i, be: (i, 0)),
            pl.BlockSpec((1, df, d), lambda i, be: (be[i], 0, 0)),
            pl.BlockSpec((1, df, d), lambda i, be: (be[i], 0, 0)),
            pl.BlockSpec((1, d, df), lambda i, be: (be[i], 0, 0)),
            pl.BlockSpec((1, 1, _BLK), lambda i, be: (i, 0, 0)),
        ],
        out_specs=pl.BlockSpec((_BLK, d), lambda i, be: (i, 0)),
    )
    ys = pl.pallas_call(
        _ffn_kernel,
        grid_spec=grid_spec,
        out_shape=jax.ShapeDtypeStruct((maxr, d), jnp.float32),
        compiler_params=pltpu.CompilerParams(
            dimension_semantics=("arbitrary",)),
    )(block_expert, xs, w1, w3, w2, rw3d)

    ysa, ysb = _sc_gather2(ys, r1, r2)

    out = pl.pallas_call(
        _add3_kernel,
        grid=(t // bts,),
        in_specs=[pl.BlockSpec((bts, d), lambda i: (i, 0))] * 3,
        out_specs=pl.BlockSpec((bts, d), lambda i: (i, 0)),
        out_shape=jax.ShapeDtypeStruct((t, d), jnp.float32),
    )(shared, ysa, ysb)

    return out.reshape(b, s, d), logits


# speedup vs baseline: 1.1557x; 1.1557x over previous
import functools

import jax
import jax.numpy as jnp
from jax import lax
from jax.experimental import pallas as pl
from jax.experimental.pallas import tpu as pltpu
from jax.experimental.pallas import tpu_sc as plsc

_NC, _NS = 2, 16
_NW = _NC * _NS
_BLK = 256


def _dot_t(a, b):
    return jax.lax.dot_general(a, b, (((1,), (1,)), ((), ())),
                               preferred_element_type=jnp.float32)


def _router_kernel(logits_ref, wmat_ref, sel_ref):
    logits = logits_ref[...]
    t, e = logits.shape
    m = jnp.max(logits, axis=-1, keepdims=True)
    ex = jnp.exp(logits - m)
    scores = ex / jnp.sum(ex, axis=-1, keepdims=True)
    lane = jax.lax.broadcasted_iota(jnp.int32, (t, e), 1)
    s1 = jnp.max(scores, axis=-1, keepdims=True)
    a1 = jnp.min(jnp.where(scores == s1, lane, e), axis=-1, keepdims=True)
    m1 = lane == a1
    scores2 = jnp.where(m1, -1.0, scores)
    s2 = jnp.max(scores2, axis=-1, keepdims=True)
    a2 = jnp.min(jnp.where(scores2 == s2, lane, e), axis=-1, keepdims=True)
    m2 = lane == a2
    denom = s1 + s2 + 1e-6
    wmat_ref[...] = (jnp.where(m1, s1, 0.0) + jnp.where(m2, s2, 0.0)) / denom
    sel_ref[...] = jnp.where(m1 | m2, 1.0, 0.0)


def _shared_kernel(x_ref, sw1_ref, sw3_ref, sw2_ref, o_ref):
    x = x_ref[...].astype(jnp.bfloat16)
    g = _dot_t(x, sw1_ref[...].astype(jnp.bfloat16))
    u = _dot_t(x, sw3_ref[...].astype(jnp.bfloat16))
    h = (g * jax.nn.sigmoid(g) * u).astype(jnp.bfloat16)
    o_ref[...] = _dot_t(h, sw2_ref[...].astype(jnp.bfloat16))


def _ffn_kernel(be_ref, xs_ref, w1_ref, w3_ref, w2_ref, rw_ref, ys_ref):
    del be_ref
    x = xs_ref[...].astype(jnp.bfloat16)
    g = _dot_t(x, w1_ref[0].astype(jnp.bfloat16))
    u = _dot_t(x, w3_ref[0].astype(jnp.bfloat16))
    h = (g * jax.nn.sigmoid(g) * u).astype(jnp.bfloat16)
    y = _dot_t(h, w2_ref[0].astype(jnp.bfloat16))
    rw = rw_ref[0, 0, :]
    ys_ref[...] = y * rw[:, None]


def _add3_kernel(a_ref, b_ref, c_ref, o_ref):
    o_ref[...] = a_ref[...] + b_ref[...] + c_ref[...]


def _sc_gather_rows(x, row_token, maxr):
    _, w = x.shape
    rows_pw = maxr // _NW
    nch = 8
    ch = rows_pw // nch
    mesh = plsc.VectorSubcoreMesh(core_axis_name="c", subcore_axis_name="s")

    @functools.partial(
        pl.kernel, mesh=mesh,
        out_type=jax.ShapeDtypeStruct((maxr, w), jnp.float32),
        scratch_types=[pltpu.VMEM((rows_pw,), jnp.int32),
                       pltpu.VMEM((ch, w), jnp.float32),
                       pltpu.VMEM((ch, w), jnp.float32),
                       pltpu.SemaphoreType.DMA,
                       pltpu.SemaphoreType.DMA])
    def k(x_hbm, tok_hbm, out_hbm, idx_v, b0, b1, s0, s1):
        wid = lax.axis_index("s") * _NC + lax.axis_index("c")
        base = wid * rows_pw
        pltpu.sync_copy(tok_hbm.at[pl.ds(base, rows_pw)], idx_v)
        bufs, sems, g = (b0, b1), (s0, s1), [None, None]
        g[0] = pltpu.async_copy(
            x_hbm.at[idx_v.at[pl.ds(0, ch)]], b0, s0)
        for i in range(nch):
            if i + 1 < nch:
                g[(i + 1) % 2] = pltpu.async_copy(
                    x_hbm.at[idx_v.at[pl.ds((i + 1) * ch, ch)]],
                    bufs[(i + 1) % 2], sems[(i + 1) % 2])
            g[i % 2].wait()
            pltpu.sync_copy(bufs[i % 2], out_hbm.at[pl.ds(base + i * ch, ch)])

    return k(x, row_token)


def _sc_gather2(ys, r1, r2):
    _, w = ys.shape
    t = r1.shape[0]
    toks_pw = t // _NW
    mesh = plsc.VectorSubcoreMesh(core_axis_name="c", subcore_axis_name="s")

    ch = toks_pw // 2

    @functools.partial(
        pl.kernel, mesh=mesh,
        out_type=(jax.ShapeDtypeStruct((t, w), jnp.float32),
                  jax.ShapeDtypeStruct((t, w), jnp.float32)),
        scratch_types=[pltpu.VMEM((toks_pw,), jnp.int32),
                       pltpu.VMEM((toks_pw,), jnp.int32),
                       pltpu.VMEM((ch, w), jnp.float32),
                       pltpu.VMEM((ch, w), jnp.float32),
                       pltpu.SemaphoreType.DMA,
                       pltpu.SemaphoreType.DMA])
    def k(ys_hbm, r1_hbm, r2_hbm, ysa_hbm, ysb_hbm,
          i1_v, i2_v, ba, bb, sa, sb):
        wid = lax.axis_index("s") * _NC + lax.axis_index("c")
        base = wid * toks_pw
        pltpu.sync_copy(r1_hbm.at[pl.ds(base, toks_pw)], i1_v)
        pltpu.sync_copy(r2_hbm.at[pl.ds(base, toks_pw)], i2_v)
        for i in range(toks_pw // ch):
            ga = pltpu.async_copy(
                ys_hbm.at[i1_v.at[pl.ds(i * ch, ch)]], ba, sa)
            gb = pltpu.async_copy(
                ys_hbm.at[i2_v.at[pl.ds(i * ch, ch)]], bb, sb)
            ga.wait()
            pltpu.sync_copy(ba, ysa_hbm.at[pl.ds(base + i * ch, ch)])
            gb.wait()
            pltpu.sync_copy(bb, ysb_hbm.at[pl.ds(base + i * ch, ch)])

    return k(ys, r1, r2)


def kernel(hidden_states, gate_w, w1, w2, w3, sw1, sw2, sw3):
    b, s, d = hidden_states.shape
    t = b * s
    e = gate_w.shape[0]
    df = w1.shape[1]
    dsh = sw1.shape[0]
    x = hidden_states.reshape(t, d)

    maxb = (t * 2) // _BLK + e
    maxr = maxb * _BLK

    logits = x @ gate_w.T
    wmat, selmask = pl.pallas_call(
        _router_kernel,
        out_shape=(jax.ShapeDtypeStruct((t, e), jnp.float32),
                   jax.ShapeDtypeStruct((t, e), jnp.float32)),
    )(logits)

    sel = selmask > 0.0
    counts = jnp.sum(selmask, axis=0).astype(jnp.int32)
    positions = (jnp.cumsum(selmask, axis=0) - selmask).astype(jnp.int32)
    pc = ((counts + _BLK - 1) // _BLK) * _BLK
    cum = jnp.cumsum(pc)
    poff = cum - pc
    dest = poff[None, :] + positions
    bi = jnp.arange(maxb, dtype=jnp.int32) * _BLK
    block_expert = jnp.minimum(
        jnp.sum((bi[:, None] >= cum[None, :]).astype(jnp.int32), axis=1),
        e - 1).astype(jnp.int32)
    dm1 = jnp.where(sel, dest, 2 * maxr)
    dm2 = jnp.where(sel, dest, -1)
    r1 = jnp.min(dm1, axis=1).astype(jnp.int32)
    r2 = jnp.max(dm2, axis=1).astype(jnp.int32)
    e1 = jnp.argmin(dm1, axis=1)
    e2 = jnp.argmax(dm2, axis=1)
    wv1 = jnp.take_along_axis(wmat, e1[:, None], axis=1)[:, 0]
    wv2 = jnp.take_along_axis(wmat, e2[:, None], axis=1)[:, 0]
    tok = jnp.arange(t, dtype=jnp.int32)
    pad_tok = (jnp.arange(maxr, dtype=jnp.int32) * 64) % t
    row_token = pad_tok.at[r1].set(tok).at[r2].set(tok)
    row_weight = jnp.zeros((maxr,), jnp.float32).at[r1].set(wv1).at[r2].set(wv2)

    xs = _sc_gather_rows(x, row_token, maxr)

    bts = 512
    shared = pl.pallas_call(
        _shared_kernel,
        grid=(t // bts,),
        in_specs=[
            pl.BlockSpec((bts, d), lambda i: (i, 0)),
            pl.BlockSpec((dsh, d), lambda i: (0, 0)),
            pl.BlockSpec((dsh, d), lambda i: (0, 0)),
            pl.BlockSpec((d, dsh), lambda i: (0, 0)),
        ],
        out_specs=pl.BlockSpec((bts, d), lambda i: (i, 0)),
        out_shape=jax.ShapeDtypeStruct((t, d), jnp.float32),
    )(x, sw1, sw3, sw2)

    rw3d = row_weight.reshape(maxb, 1, _BLK)
    grid_spec = pltpu.PrefetchScalarGridSpec(
        num_scalar_prefetch=1,
        grid=(maxb,),
        in_specs=[
            pl.BlockSpec((_BLK, d), lambda i, be: (i, 0)),
            pl.BlockSpec((1, df, d), lambda i, be: (be[i], 0, 0)),
            pl.BlockSpec((1, df, d), lambda i, be: (be[i], 0, 0)),
            pl.BlockSpec((1, d, df), lambda i, be: (be[i], 0, 0)),
            pl.BlockSpec((1, 1, _BLK), lambda i, be: (i, 0, 0)),
        ],
        out_specs=pl.BlockSpec((_BLK, d), lambda i, be: (i, 0)),
    )
    ys = pl.pallas_call(
        _ffn_kernel,
        grid_spec=grid_spec,
        out_shape=jax.ShapeDtypeStruct((maxr, d), jnp.float32),
        compiler_params=pltpu.CompilerParams(
            dimension_semantics=("arbitrary",)),
    )(block_expert, xs, w1, w3, w2, rw3d)

    ysa, ysb = _sc_gather2(ys, r1, r2)

    out = pl.pallas_call(
        _add3_kernel,
        grid=(t // bts,),
        in_specs=[pl.BlockSpec((bts, d), lambda i: (i, 0))] * 3,
        out_specs=pl.BlockSpec((bts, d), lambda i: (i, 0)),
        out_shape=jax.ShapeDtypeStruct((t, d), jnp.float32),
    )(shared, ysa, ysb)

    return out.reshape(b, s, d), logits
